# baseline (speedup 1.0000x reference)
import jax
import jax.numpy as jnp
from jax import lax
from jax.experimental import pallas as pl
from jax.experimental.pallas import tpu as pltpu

_B, _T, _C = 8, 4096, 64
_P = 4
_D = 256
_K = 1024
_N = (_B * _T) // _P
_BLK = 1024
_GRID = _N // _BLK
_H = 2
_TR = _T // 2


def _vq_body(x_ref, we_ref, be_ref, cb_ref, wd_ref, bd_ref, out_ref,
             dec0, dec1, e2_ref):
    i = pl.program_id(0)
    decs = [dec0, dec1]

    @pl.when(i == 0)
    def _():
        cb = cb_ref[...]
        for h in range(_H):
            decs[h][...] = (
                jnp.dot(cb, wd_ref[:, h * 128:(h + 1) * 128],
                        preferred_element_type=jnp.float32)
                + bd_ref[:, h * 128:(h + 1) * 128]
            )
        e2_ref[...] = jnp.sum(cb * cb, axis=1, keepdims=True).T

    x = jnp.concatenate(
        [x_ref[0, pl.Slice(h, _BLK, _H), :] for h in range(_H)], axis=1)
    z = jnp.dot(x, we_ref[...], preferred_element_type=jnp.float32) + be_ref[...]
    s = lax.dot_general(
        z, cb_ref[...], (((1,), (1,)), ((), ())),
        preferred_element_type=jnp.float32)
    z2 = jnp.sum(z * z, axis=1, keepdims=True)
    dists = z2 - 2.0 * s + e2_ref[...]
    minval = jnp.min(dists, axis=1, keepdims=True)
    iota = lax.broadcasted_iota(jnp.int32, (_BLK, _K), 1)
    idx = jnp.min(jnp.where(dists == minval, iota, _K), axis=1)
    one_hot = (iota == idx[:, None]).astype(jnp.float32)
    for h in range(_H):
        q_h = jnp.dot(one_hot, decs[h][...],
                      preferred_element_type=jnp.float32)
        out_ref[0, pl.Slice(h, _BLK, _H), :] = q_h


def kernel(brain_wave, W_enc, b_enc, codebook, W_dec, b_dec):
    x = brain_wave.reshape(_B, _TR, 2 * _C)
    out = pl.pallas_call(
        _vq_body,
        grid=(_GRID,),
        in_specs=[
            pl.BlockSpec((1, _TR, 2 * _C), lambda i: (i, 0, 0)),
            pl.BlockSpec((_P * _C, _D), lambda i: (0, 0)),
            pl.BlockSpec((1, _D), lambda i: (0, 0)),
            pl.BlockSpec((_K, _D), lambda i: (0, 0)),
            pl.BlockSpec((_D, _P * _C), lambda i: (0, 0)),
            pl.BlockSpec((1, _P * _C), lambda i: (0, 0)),
        ],
        out_specs=pl.BlockSpec((1, _TR, 2 * _C), lambda i: (i, 0, 0)),
        out_shape=jax.ShapeDtypeStruct((_B, _TR, 2 * _C), jnp.float32),
        scratch_shapes=[
            pltpu.VMEM((_K, 128), jnp.float32),
            pltpu.VMEM((_K, 128), jnp.float32),
            pltpu.VMEM((1, _K), jnp.float32),
        ],
    )(x, W_enc, b_enc.reshape(1, _D), codebook, W_dec,
      b_dec.reshape(1, _P * _C))
    return out.reshape(_B, _T, _C)


# speedup vs baseline: 1.3270x; 1.3270x over previous
import jax
import jax.numpy as jnp
from jax import lax
from jax.experimental import pallas as pl
from jax.experimental.pallas import tpu as pltpu

_B, _T, _C = 8, 4096, 64
_P = 4
_D = 256
_K = 1024
_N = (_B * _T) // _P
_BLK = 512
_GRID = _N // _BLK
_SPB = _T // (_P * _BLK)


def _vq_body(x_ref, we_ref, be_ref, cb_ref, wd_ref, bd_ref, out_ref,
             dec0, dec1, dec2, dec3, e2_ref):
    i = pl.program_id(0)
    decs = [dec0, dec1, dec2, dec3]

    @pl.when(i == 0)
    def _():
        cb = cb_ref[...]
        for r in range(_P):
            decs[r][...] = (
                jnp.dot(cb, wd_ref[:, r * _C:(r + 1) * _C],
                        preferred_element_type=jnp.float32)
                + bd_ref[:, r * _C:(r + 1) * _C]
            )
        e2_ref[...] = jnp.sum(cb * cb, axis=1, keepdims=True).T

    x = jnp.concatenate(
        [x_ref[0, pl.Slice(r, _BLK, _P), :] for r in range(_P)], axis=1)
    z = jnp.dot(x, we_ref[...], preferred_element_type=jnp.float32) + be_ref[...]
    s = lax.dot_general(
        z, cb_ref[...], (((1,), (1,)), ((), ())),
        preferred_element_type=jnp.float32)
    z2 = jnp.sum(z * z, axis=1, keepdims=True)
    dists = z2 - 2.0 * s + e2_ref[...]
    minval = jnp.min(dists, axis=1, keepdims=True)
    iota = lax.broadcasted_iota(jnp.int32, (_BLK, _K), 1)
    idx = jnp.min(jnp.where(dists == minval, iota, _K), axis=1)
    one_hot = (iota == idx[:, None]).astype(jnp.float32)
    for r in range(_P):
        q_r = jnp.dot(one_hot, decs[r][...],
                      preferred_element_type=jnp.float32)
        out_ref[0, pl.Slice(r, _BLK, _P), :] = q_r


def kernel(brain_wave, W_enc, b_enc, codebook, W_dec, b_dec):
    out = pl.pallas_call(
        _vq_body,
        grid=(_GRID,),
        in_specs=[
            pl.BlockSpec((1, _P * _BLK, _C),
                         lambda i: (i // _SPB, i % _SPB, 0)),
            pl.BlockSpec((_P * _C, _D), lambda i: (0, 0)),
            pl.BlockSpec((1, _D), lambda i: (0, 0)),
            pl.BlockSpec((_K, _D), lambda i: (0, 0)),
            pl.BlockSpec((_D, _P * _C), lambda i: (0, 0)),
            pl.BlockSpec((1, _P * _C), lambda i: (0, 0)),
        ],
        out_specs=pl.BlockSpec((1, _P * _BLK, _C),
                               lambda i: (i // _SPB, i % _SPB, 0)),
        out_shape=jax.ShapeDtypeStruct((_B, _T, _C), jnp.float32),
        scratch_shapes=[
            pltpu.VMEM((_K, _C), jnp.float32),
            pltpu.VMEM((_K, _C), jnp.float32),
            pltpu.VMEM((_K, _C), jnp.float32),
            pltpu.VMEM((_K, _C), jnp.float32),
            pltpu.VMEM((1, _K), jnp.float32),
        ],
    )(brain_wave, W_enc, b_enc.reshape(1, _D), codebook, W_dec,
      b_dec.reshape(1, _P * _C))
    return out


# speedup vs baseline: 1.4724x; 1.1096x over previous
import jax
import jax.numpy as jnp
from jax import lax
from jax.experimental import pallas as pl
from jax.experimental.pallas import tpu as pltpu

_B, _T, _C = 8, 4096, 64
_P = 4
_D = 256
_K = 1024
_N = (_B * _T) // _P
_BLK = 2048
_GRID = _N // _BLK
_BPS = (_P * _BLK) // _T


def _vq_body(x_ref, we_ref, be_ref, cb_ref, wd_ref, bd_ref, out_ref,
             dec0, dec1, dec2, dec3, e2_ref):
    i = pl.program_id(0)
    decs = [dec0, dec1, dec2, dec3]

    @pl.when(i == 0)
    def _():
        cb = cb_ref[...]
        for r in range(_P):
            decs[r][...] = (
                jnp.dot(cb, wd_ref[:, r * _C:(r + 1) * _C],
                        preferred_element_type=jnp.float32)
                + bd_ref[:, r * _C:(r + 1) * _C]
            )
        e2_ref[...] = jnp.sum(cb * cb, axis=1, keepdims=True).T

    x = jnp.concatenate([
        jnp.concatenate(
            [x_ref[b, pl.Slice(r, _T // _P, _P), :] for r in range(_P)],
            axis=1)
        for b in range(_BPS)], axis=0)
    z = jnp.dot(x, we_ref[...], preferred_element_type=jnp.float32) + be_ref[...]
    s = lax.dot_general(
        z, cb_ref[...], (((1,), (1,)), ((), ())),
        preferred_element_type=jnp.float32)
    z2 = jnp.sum(z * z, axis=1, keepdims=True)
    dists = z2 - 2.0 * s + e2_ref[...]
    minval = jnp.min(dists, axis=1, keepdims=True)
    iota = lax.broadcasted_iota(jnp.int32, (_BLK, _K), 1)
    idx = jnp.min(jnp.where(dists == minval, iota, _K), axis=1)
    one_hot = (iota == idx[:, None]).astype(jnp.float32)
    for r in range(_P):
        q_r = jnp.dot(one_hot, decs[r][...],
                      preferred_element_type=jnp.float32)
        for b in range(_BPS):
            out_ref[b, pl.Slice(r, _T // _P, _P), :] = (
                q_r[b * (_T // _P):(b + 1) * (_T // _P), :])


def kernel(brain_wave, W_enc, b_enc, codebook, W_dec, b_dec):
    out = pl.pallas_call(
        _vq_body,
        grid=(_GRID,),
        in_specs=[
            pl.BlockSpec((_BPS, _T, _C), lambda i: (i, 0, 0)),
            pl.BlockSpec((_P * _C, _D), lambda i: (0, 0)),
            pl.BlockSpec((1, _D), lambda i: (0, 0)),
            pl.BlockSpec((_K, _D), lambda i: (0, 0)),
            pl.BlockSpec((_D, _P * _C), lambda i: (0, 0)),
            pl.BlockSpec((1, _P * _C), lambda i: (0, 0)),
        ],
        out_specs=pl.BlockSpec((_BPS, _T, _C), lambda i: (i, 0, 0)),
        out_shape=jax.ShapeDtypeStruct((_B, _T, _C), jnp.float32),
        scratch_shapes=[
            pltpu.VMEM((_K, _C), jnp.float32),
            pltpu.VMEM((_K, _C), jnp.float32),
            pltpu.VMEM((_K, _C), jnp.float32),
            pltpu.VMEM((_K, _C), jnp.float32),
            pltpu.VMEM((1, _K), jnp.float32),
        ],
    )(brain_wave, W_enc, b_enc.reshape(1, _D), codebook, W_dec,
      b_dec.reshape(1, _P * _C))
    return out
